# baseline (speedup 1.0000x reference)
import functools

import jax
import jax.numpy as jnp
from jax import lax
from jax.experimental import pallas as pl
from jax.experimental.pallas import tpu as pltpu
from jax.experimental.pallas import tpu_sc as plsc

B = 128
N = 100000
L = 16
NW = 32
ROWS_PER_W = B // NW
U = 5
NCH = 5
CH_V = N // (NCH * L)
CH = CH_V * L
CAPV = 64

_mesh = plsc.VectorSubcoreMesh(core_axis_name="c", subcore_axis_name="s")


def _bcast_last(v, scratch):
    scratch[...] = v
    return plsc.load_gather(scratch, [jnp.full((L,), L - 1, jnp.int32)])


def _allsum(v, scratch):
    return _bcast_last(plsc.cumsum(v), scratch)


def _allmax(v, scratch):
    return _bcast_last(plsc.cummax(v), scratch)


@functools.partial(
    pl.kernel,
    out_type=jax.ShapeDtypeStruct((NW, L), jnp.float32),
    scratch_types=[
        pltpu.VMEM((CH,), jnp.float32),
        pltpu.VMEM((CH,), jnp.float32),
        pltpu.VMEM((CAPV * L,), jnp.float32),
        pltpu.VMEM((B,), jnp.int32),
        pltpu.VMEM((L,), jnp.float32),
        pltpu.VMEM((L,), jnp.float32),
        pltpu.VMEM((L,), jnp.int32),
        pltpu.SemaphoreType.DMA,
        pltpu.SemaphoreType.DMA,
    ],
    mesh=_mesh,
    compiler_params=pltpu.CompilerParams(needs_layout_passes=False),
)
def _sc_loss(x_hbm, t_hbm, out_hbm, buf0, buf1, cand_v, targ_v, acc_v,
             red_f, red_i, sem0, sem1):
    wid = lax.axis_index("c") * 16 + lax.axis_index("s")
    lanes = lax.iota(jnp.int32, L)
    bufs = (buf0, buf1)
    sems = (sem0, sem1)
    pltpu.sync_copy(t_hbm, targ_v)

    def pass_skq_ref(ref, nvec, tau, init):
        def body(i, c):
            s, k, q = c
            v = ref[pl.ds(i * L, L)]
            sv = jnp.where(v > tau, v, 0.0)
            return (s + sv, k + jnp.where(v > tau, 1.0, 0.0), q + sv * v)

        return lax.fori_loop(0, nvec, body, init)

    acc = jnp.zeros((L,), jnp.float32)
    for r in range(ROWS_PER_W):
        row = wid * ROWS_PER_W + r

        blk = (row // L) * L
        tvec = targ_v[pl.ds(blk, L)]
        tg = _allsum(jnp.where(lanes == row - blk, tvec, 0), red_i)

        neg = jnp.full((L,), -jnp.inf, jnp.float32)
        cur = jnp.zeros((L,), jnp.int32)
        x_t = jnp.zeros((L,), jnp.float32)
        sofar = neg
        handles = [None] * NCH
        handles[0] = pltpu.async_copy(
            x_hbm.at[pl.ds(row * N, CH)], bufs[0], sems[0])

        for c in range(NCH):
            handles[c].wait()
            if c + 1 < NCH:
                handles[c + 1] = pltpu.async_copy(
                    x_hbm.at[pl.ds(row * N + (c + 1) * CH, CH)],
                    bufs[(c + 1) % 2], sems[(c + 1) % 2])
            buf = bufs[c % 2]

            if c == 0:
                def max_body(i, ms, buf=buf):
                    return tuple(
                        jnp.maximum(ms[u], buf[pl.ds(i * (U * L) + u * L, L)])
                        for u in range(U))

                ms = lax.fori_loop(0, CH_V // U, max_body, (neg,) * U)
                m16 = ms[0]
                for u in range(1, U):
                    m16 = jnp.maximum(m16, ms[u])
                sofar = _allmax(m16, red_f)
                thr = sofar - 1.0

                def comp_body(i, cur, buf=buf, thr=thr):
                    for u in range(U):
                        v = buf[pl.ds(i * (U * L) + u * L, L)]
                        keep = v > thr
                        slot = jnp.minimum(cur, CAPV - 1) * L + lanes
                        plsc.store_scatter(cand_v, [slot], v,
                                           mask=keep & (cur < CAPV))
                        cur = cur + jnp.where(keep, 1, 0)
                    return cur

                cur = lax.fori_loop(0, CH_V // U, comp_body, cur)
            else:
                def fused_body(i, carry, buf=buf):
                    cur, rms = carry
                    rms = list(rms)
                    for u in range(U):
                        v = buf[pl.ds(i * (U * L) + u * L, L)]
                        rms[u] = jnp.maximum(rms[u], v)
                        keep = v > rms[u] - 1.0
                        slot = jnp.minimum(cur, CAPV - 1) * L + lanes
                        plsc.store_scatter(cand_v, [slot], v,
                                           mask=keep & (cur < CAPV))
                        cur = cur + jnp.where(keep, 1, 0)
                    return cur, tuple(rms)

                cur, rms = lax.fori_loop(0, CH_V // U, fused_body,
                                         (cur, (sofar,) * U))
                m16 = rms[0]
                for u in range(1, U):
                    m16 = jnp.maximum(m16, rms[u])
                sofar = _allmax(m16, red_f)

            in_chunk = jnp.any((tg >= c * CH) & (tg < (c + 1) * CH))
            x_t = lax.cond(
                in_chunk,
                lambda buf=buf, c=c: plsc.load_gather(buf, [tg - c * CH]),
                lambda: x_t)

        big = sofar
        tau0 = big - 1.0

        def michelot_cand():
            def pass_skq(tau):
                def body(i, cacc):
                    s, k, q = cacc
                    v = cand_v[pl.ds(i * L, L)]
                    m = (v > tau) & (cur > i)
                    sv = jnp.where(m, v, 0.0)
                    return (s + sv, k + jnp.where(m, 1.0, 0.0), q + sv * v)

                z = jnp.zeros((L,), jnp.float32)
                s, k, q = lax.fori_loop(0, CAPV, body, (z, z, z))
                return _allsum(s, red_f), _allsum(k, red_f), _allsum(q, red_f)

            s, k, q = pass_skq(tau0)
            carry = (tau0, (s - 1.0) / k, jnp.int32(1), s, k, q)

            def w_cond(ca):
                tau, new_tau, it, _, _, _ = ca
                return jnp.all(new_tau > tau) & (it < 64)

            def w_body(ca):
                _, tau, it, _, _, _ = ca
                s, k, q = pass_skq(tau)
                return (tau, (s - 1.0) / k, it + 1, s, k, q)

            _, tau, _, s, k, q = lax.while_loop(w_cond, w_body, carry)
            return tau, s, k, q

        def michelot_stream():
            def pass_skq(tau):
                z = jnp.zeros((L,), jnp.float32)
                acc3 = (z, z, z)
                for c in range(NCH):
                    pltpu.sync_copy(x_hbm.at[pl.ds(row * N + c * CH, CH)], buf0)
                    acc3 = pass_skq_ref(buf0, CH_V, tau, acc3)
                s, k, q = acc3
                return (_allsum(s, red_f), _allsum(k, red_f),
                        _allsum(q, red_f))

            s, k, q = pass_skq(tau0)
            carry = (tau0, (s - 1.0) / k, jnp.int32(1), s, k, q)

            def w_cond(ca):
                tau, new_tau, it, _, _, _ = ca
                return jnp.all(new_tau > tau) & (it < 64)

            def w_body(ca):
                _, tau, it, _, _, _ = ca
                s, k, q = pass_skq(tau)
                return (tau, (s - 1.0) / k, it + 1, s, k, q)

            _, tau, _, s, k, q = lax.while_loop(w_cond, w_body, carry)
            return tau, s, k, q

        overflow = jnp.any(cur > CAPV)
        tau, s, k, q = lax.cond(overflow, michelot_stream, michelot_cand)

        sum_px = q - tau * s
        sum_p2 = q - 2.0 * tau * s + k * tau * tau
        loss = (1.0 - sum_p2) * 0.5 + sum_px - x_t
        acc = acc + jnp.where(lanes == r, loss, 0.0)

    acc_v[...] = acc
    pltpu.sync_copy(acc_v, out_hbm.at[wid])


def kernel(X, target):
    part = _sc_loss(X.reshape(-1), target.astype(jnp.int32))
    return jnp.sum(part) / jnp.float32(B)
